# baseline (speedup 1.0000x reference)
import jax
import jax.numpy as jnp
from jax import lax
from jax.experimental import pallas as pl
from jax.experimental.pallas import tpu as pltpu
from jax.experimental.pallas import tpu_sc as plsc

_NC = 2
_NS = 16
_NW = _NC * _NS

_SEQ = 4096
_D = 2048
_RPW = _SEQ // _NW
_CH = 4
_NCHUNK = _RPW // _CH
_NV = _CH * _D // 16
_DEPTH = 3


def _sc_body(x_hbm, pos_hbm, out_hbm, pb, x0, x1, sem_in, sem_out):
    wid = lax.axis_index("s") * _NC + lax.axis_index("c")
    base = wid * _RPW

    def start_in(i, p):
        r0 = base + i * _CH
        return [
            pltpu.async_copy(pos_hbm.at[pl.ds(r0, _CH), :], pb.at[p],
                             sem_in.at[p]),
            pltpu.async_copy(x_hbm.at[0, pl.ds(r0, _CH), :], x0.at[p],
                             sem_in.at[p]),
            pltpu.async_copy(x_hbm.at[1, pl.ds(r0, _CH), :], x1.at[p],
                             sem_in.at[p]),
        ]

    def start_out(i, p):
        r0 = base + i * _CH
        return [
            pltpu.async_copy(x0.at[p], out_hbm.at[0, pl.ds(r0, _CH), :],
                             sem_out.at[p]),
            pltpu.async_copy(x1.at[p], out_hbm.at[1, pl.ds(r0, _CH), :],
                             sem_out.at[p]),
        ]

    def compute(p):
        @plsc.parallel_loop(0, _NV, 1, unroll=8)
        def body(j):
            r = j >> 7
            c = (j & 127) * 16
            s = pl.ds(c, 16)
            pv = pb[p, r, s]
            plsc.addupdate(x0.at[p, r, s], pv)
            plsc.addupdate(x1.at[p, r, s], pv)

    in_d = {0: start_in(0, 0), 1: start_in(1, 1)}
    out_d = {}
    for i in range(_NCHUNK):
        p = i % _DEPTH
        for d in in_d[i]:
            d.wait()
        compute(p)
        out_d[i] = start_out(i, p)
        if i + 2 < _NCHUNK:
            if i >= 1:
                for d in out_d[i - 1]:
                    d.wait()
            in_d[i + 2] = start_in(i + 2, (i + 2) % _DEPTH)
    for i in range(_NCHUNK - 3, _NCHUNK):
        for d in out_d[i]:
            d.wait()


def kernel(x, pos_table):
    b, s, d = x.shape
    mesh = plsc.VectorSubcoreMesh(core_axis_name="c", subcore_axis_name="s")
    return pl.kernel(
        _sc_body,
        out_type=jax.ShapeDtypeStruct((b, s, d), x.dtype),
        mesh=mesh,
        scratch_types=[
            pltpu.VMEM((_DEPTH, _CH, _D), jnp.float32),
            pltpu.VMEM((_DEPTH, _CH, _D), jnp.float32),
            pltpu.VMEM((_DEPTH, _CH, _D), jnp.float32),
            pltpu.SemaphoreType.DMA((_DEPTH,)),
            pltpu.SemaphoreType.DMA((_DEPTH,)),
        ],
    )(x, pos_table)


# speedup vs baseline: 1.0341x; 1.0341x over previous
import jax
import jax.numpy as jnp
from jax import lax
from jax.experimental import pallas as pl
from jax.experimental.pallas import tpu as pltpu
from jax.experimental.pallas import tpu_sc as plsc

_NC = 2
_NS = 16
_NW = _NC * _NS

_SEQ = 4096
_D = 2048
_RPW = _SEQ // _NW
_CH = 8
_NCHUNK = _RPW // _CH
_NV = _CH * _D // 16


def _sc_body(x_hbm, pos_hbm, out_hbm, pb, x0, x1, sem_in, sem_out):
    wid = lax.axis_index("s") * _NC + lax.axis_index("c")
    base = wid * _RPW

    def start_in(i, p):
        r0 = base + i * _CH
        return [
            pltpu.async_copy(pos_hbm.at[pl.ds(r0, _CH), :], pb.at[p],
                             sem_in.at[p]),
            pltpu.async_copy(x_hbm.at[0, pl.ds(r0, _CH), :], x0.at[p],
                             sem_in.at[p]),
            pltpu.async_copy(x_hbm.at[1, pl.ds(r0, _CH), :], x1.at[p],
                             sem_in.at[p]),
        ]

    def start_out(i, p):
        r0 = base + i * _CH
        return [
            pltpu.async_copy(x0.at[p], out_hbm.at[0, pl.ds(r0, _CH), :],
                             sem_out.at[p]),
            pltpu.async_copy(x1.at[p], out_hbm.at[1, pl.ds(r0, _CH), :],
                             sem_out.at[p]),
        ]

    def compute(p):
        @plsc.parallel_loop(0, _NV, 1, unroll=8)
        def body(j):
            r = j >> 7
            c = (j & 127) * 16
            s = pl.ds(c, 16)
            pv = pb[p, r, s]
            plsc.addupdate(x0.at[p, r, s], pv)
            plsc.addupdate(x1.at[p, r, s], pv)

    in_d = {0: start_in(0, 0)}
    out_d = {}
    for i in range(_NCHUNK):
        p = i % 2
        if i + 1 < _NCHUNK:
            if i >= 1:
                for d in out_d[i - 1]:
                    d.wait()
            in_d[i + 1] = start_in(i + 1, (i + 1) % 2)
        for d in in_d[i]:
            d.wait()
        compute(p)
        out_d[i] = start_out(i, p)
    for d in out_d[_NCHUNK - 2] + out_d[_NCHUNK - 1]:
        d.wait()


def kernel(x, pos_table):
    b, s, d = x.shape
    mesh = plsc.VectorSubcoreMesh(core_axis_name="c", subcore_axis_name="s")
    return pl.kernel(
        _sc_body,
        out_type=jax.ShapeDtypeStruct((b, s, d), x.dtype),
        mesh=mesh,
        scratch_types=[
            pltpu.VMEM((2, _CH, _D), jnp.float32),
            pltpu.VMEM((2, _CH, _D), jnp.float32),
            pltpu.VMEM((2, _CH, _D), jnp.float32),
            pltpu.SemaphoreType.DMA((2,)),
            pltpu.SemaphoreType.DMA((2,)),
        ],
    )(x, pos_table)
